# baseline (speedup 1.0000x reference)
import functools

import jax
import jax.numpy as jnp
from jax import lax
from jax.experimental import pallas as pl
from jax.experimental.pallas import tpu as pltpu
from jax.experimental.pallas import tpu_sc as plsc

N_NODES = 10000
SEQ_LEN = 32
IN_CH = 128
OUT_CH = 128

NUM_CORES = 2
NUM_SUBCORES = 16
NUM_WORKERS = NUM_CORES * NUM_SUBCORES
NODES_PAD = 10240
NPW = NODES_PAD // NUM_WORKERS
LAST_VALID = N_NODES - (NUM_WORKERS - 1) * NPW
N_SEMS = 4


def _zmm_body(x_ref, wt_ref, bs_ref, z_ref):
    z_ref[0] = (
        lax.dot_general(
            x_ref[...].astype(jnp.bfloat16), wt_ref[0].astype(jnp.bfloat16),
            (((1,), (0,)), ((), ())),
            preferred_element_type=jnp.float32,
        )
        + bs_ref[...]
    )


def _tc_zmm(x, Wt, bs):
    return pl.pallas_call(
        _zmm_body,
        grid=(SEQ_LEN,),
        in_specs=[
            pl.BlockSpec((N_NODES, IN_CH), lambda s: (0, 0)),
            pl.BlockSpec((1, IN_CH, OUT_CH), lambda s: (s, 0, 0)),
            pl.BlockSpec((1, OUT_CH), lambda s: (0, 0)),
        ],
        out_specs=pl.BlockSpec((1, N_NODES, OUT_CH), lambda s: (s, 0, 0)),
        out_shape=jax.ShapeDtypeStruct((SEQ_LEN, N_NODES, OUT_CH),
                                       jnp.float32),
    )(x, Wt, bs)


def _sc_acc_body(z_hbm, idx_hbm, out_hbm, idx_all, acc, *sems):
    wid = lax.axis_index("s") * NUM_CORES + lax.axis_index("c")

    pltpu.sync_copy(idx_hbm.at[pl.ds(wid * SEQ_LEN * NPW, SEQ_LEN * NPW)],
                    idx_all)

    pltpu.sync_copy(z_hbm.at[idx_all.at[pl.ds(0, NPW)]], acc)
    descs = []
    for s in range(1, SEQ_LEN):
        descs.append(pltpu.async_copy(
            z_hbm.at[idx_all.at[pl.ds(s * NPW, NPW)]], acc,
            sems[s % N_SEMS], add=True))
    for d in descs:
        d.wait()

    @pl.when(wid < NUM_WORKERS - 1)
    def _full():
        pltpu.sync_copy(acc, out_hbm.at[pl.ds(wid * NPW, NPW)])

    @pl.when(wid == NUM_WORKERS - 1)
    def _tail():
        pltpu.sync_copy(acc.at[pl.ds(0, LAST_VALID)],
                        out_hbm.at[pl.ds((NUM_WORKERS - 1) * NPW, LAST_VALID)])


def _sc_acc(z_flat, idx_flat):
    mesh = plsc.VectorSubcoreMesh(core_axis_name="c", subcore_axis_name="s")
    kfn = pl.kernel(
        _sc_acc_body,
        mesh=mesh,
        out_type=jax.ShapeDtypeStruct((N_NODES, OUT_CH), jnp.float32),
        scratch_types=(
            [pltpu.VMEM((SEQ_LEN * NPW,), jnp.int32),
             pltpu.VMEM((NPW, OUT_CH), jnp.float32)]
            + [pltpu.SemaphoreType.DMA] * N_SEMS
        ),
    )
    return kfn(z_flat, idx_flat)


@jax.jit
def kernel(x, indices, W, b):
    idx32 = indices.astype(jnp.int32)
    idxp = jnp.pad(idx32, ((0, NODES_PAD - N_NODES), (0, 0)))
    offs = jnp.arange(SEQ_LEN, dtype=jnp.int32) * N_NODES
    idxa = idxp + offs[None, :]
    idx_flat = (idxa.T.reshape(SEQ_LEN, NUM_WORKERS, NPW)
                .transpose(1, 0, 2).reshape(-1))
    Wt = W.reshape(OUT_CH, SEQ_LEN, IN_CH).transpose(1, 2, 0)
    bs = (b / SEQ_LEN).reshape(1, OUT_CH)
    z = _tc_zmm(x, Wt, bs)
    z_flat = z.reshape(SEQ_LEN * N_NODES, OUT_CH)
    return _sc_acc(z_flat, idx_flat)


# speedup vs baseline: 1.2839x; 1.2839x over previous
import functools

import jax
import jax.numpy as jnp
from jax import lax
from jax.experimental import pallas as pl
from jax.experimental.pallas import tpu as pltpu
from jax.experimental.pallas import tpu_sc as plsc

N_NODES = 10000
SEQ_LEN = 32
IN_CH = 128
OUT_CH = 128

NUM_CORES = 2
NUM_SUBCORES = 16
NUM_WORKERS = NUM_CORES * NUM_SUBCORES
ROWS_PER_WORKER = N_NODES

CHUNK = 400
N_CHUNKS = ROWS_PER_WORKER // CHUNK
NBUF = 2


def _sc_gather_body(table_hbm, idx_hbm, out_hbm, idx_all, *bufs):
    rows_v = bufs[:NBUF]
    gsem = bufs[NBUF:2 * NBUF]
    wsem = bufs[2 * NBUF:3 * NBUF]
    wid = lax.axis_index("s") * NUM_CORES + lax.axis_index("c")
    base = wid * ROWS_PER_WORKER

    pltpu.sync_copy(idx_hbm.at[pl.ds(base, ROWS_PER_WORKER)], idx_all)

    def start_gather(c):
        b = c % NBUF
        pltpu.make_async_copy(
            table_hbm.at[idx_all.at[pl.ds(c * CHUNK, CHUNK)]],
            rows_v[b], gsem[b]).start()

    for c in range(NBUF):
        start_gather(c)
    for c in range(N_CHUNKS):
        b = c % NBUF
        pltpu.make_async_copy(
            table_hbm.at[idx_all.at[pl.ds(c * CHUNK, CHUNK)]],
            rows_v[b], gsem[b]).wait()
        wb = pltpu.make_async_copy(
            rows_v[b], out_hbm.at[pl.ds(base + c * CHUNK, CHUNK)], wsem[b])
        wb.start()
        if c + NBUF < N_CHUNKS:
            wb.wait()
            start_gather(c + NBUF)
        else:
            wb.wait()


def _sc_gather(x, idx_flat):
    mesh = plsc.VectorSubcoreMesh(core_axis_name="c", subcore_axis_name="s")
    kfn = pl.kernel(
        _sc_gather_body,
        mesh=mesh,
        out_type=jax.ShapeDtypeStruct((SEQ_LEN * N_NODES, IN_CH), jnp.float32),
        scratch_types=(
            [pltpu.VMEM((ROWS_PER_WORKER,), jnp.int32)]
            + [pltpu.VMEM((CHUNK, IN_CH), jnp.float32)] * NBUF
            + [pltpu.SemaphoreType.DMA] * (2 * NBUF)
        ),
    )
    return kfn(x, idx_flat)


def _mm_body(g_ref, wt_ref, b_ref, o_ref):
    acc = jnp.broadcast_to(b_ref[...], o_ref.shape)
    for s in range(SEQ_LEN):
        acc = acc + lax.dot_general(
            g_ref[s].astype(jnp.bfloat16), wt_ref[s].astype(jnp.bfloat16),
            (((1,), (0,)), ((), ())),
            preferred_element_type=jnp.float32,
        )
    o_ref[...] = acc


def _tc_matmul(gout, Wt, b):
    m_block = 1000
    grid = (N_NODES // m_block,)
    return pl.pallas_call(
        _mm_body,
        grid=grid,
        in_specs=[
            pl.BlockSpec((SEQ_LEN, m_block, IN_CH), lambda i: (0, i, 0)),
            pl.BlockSpec((SEQ_LEN, IN_CH, OUT_CH), lambda i: (0, 0, 0)),
            pl.BlockSpec((1, OUT_CH), lambda i: (0, 0)),
        ],
        out_specs=pl.BlockSpec((m_block, OUT_CH), lambda i: (i, 0)),
        out_shape=jax.ShapeDtypeStruct((N_NODES, OUT_CH), jnp.float32),
    )(gout, Wt, b)


@jax.jit
def kernel(x, indices, W, b):
    idx_flat = indices.astype(jnp.int32).T.reshape(-1)
    Wt = W.reshape(OUT_CH, SEQ_LEN, IN_CH).transpose(1, 2, 0)
    g = _sc_gather(x, idx_flat)
    gout = g.reshape(SEQ_LEN, N_NODES, IN_CH)
    return _tc_matmul(gout, Wt, b.reshape(1, OUT_CH))
